# baseline (speedup 1.0000x reference)
import functools

import jax
import jax.numpy as jnp
from jax import lax
from jax.experimental import pallas as pl
from jax.experimental.pallas import tpu as pltpu
from jax.experimental.pallas import tpu_sc as plsc

N_NODES = 10000
N_EDGES = 320000
H = 64
EPS = 1e-5

NC = 2
NS = 16
NW = NC * NS
EPW = N_EDGES // NW
GB = 80
NCHUNK = EPW // GB

NR = 320
N_PAD = NW * NR
DC = 8000
NDC = N_EDGES // DC
GR = 64

_mesh = None


def _get_mesh():
    global _mesh
    if _mesh is None:
        _mesh = plsc.VectorSubcoreMesh(core_axis_name="c", subcore_axis_name="s")
    return _mesh


_PREC = jax.lax.Precision.HIGHEST


def _dot(a, b):
    return jnp.dot(a, b, precision=_PREC)


def _ab_body(posp_ref, x_ref, wd_ref, wb_ref, wc_ref, b1_ref, t_ref):
    p = posp_ref[...]
    t_ref[:, 0:H] = _dot(p, wd_ref[...]) + _dot(x_ref[...], wc_ref[...]) + b1_ref[...]
    t_ref[:, H:2 * H] = _dot(p, wb_ref[...])


def _tables(posp, x, wd, wb, wc, b1):
    return pl.pallas_call(
        _ab_body,
        out_shape=jax.ShapeDtypeStruct((N_NODES, 2 * H), jnp.float32),
    )(posp, x, wd, wb, wc, b1[None, :])


def _gather_body(t_hbm, dst_hbm, src_hbm, r1_hbm, st_hbm,
                 didx, sidx, ab0, ab1, bb0, bb1, rbuf, stat_v, sem0, sem1):
    c = lax.axis_index("c")
    s = lax.axis_index("s")
    wid = s * NC + c
    base = wid * EPW
    pltpu.sync_copy(dst_hbm.at[pl.ds(base, EPW)], didx)
    pltpu.sync_copy(src_hbm.at[pl.ds(base, EPW)], sidx)

    abufs = (ab0, ab1)
    bbufs = (bb0, bb1)
    sems = (sem0, sem1)

    def fire(j, b):
        pltpu.async_copy(t_hbm.at[didx.at[pl.ds(j * GB, GB)]], abufs[b], sems[b])
        pltpu.async_copy(t_hbm.at[sidx.at[pl.ds(j * GB, GB)]], bbufs[b], sems[b])

    def drain(b):
        pltpu.make_async_copy(t_hbm.at[didx.at[pl.ds(0, GB)]], abufs[b], sems[b]).wait()
        pltpu.make_async_copy(t_hbm.at[sidx.at[pl.ds(0, GB)]], bbufs[b], sems[b]).wait()

    def compute(j, b, st):
        ab = abufs[b]
        bb = bbufs[b]

        def row(i, st):
            s0, s1, s2, s3, q0, q1, q2, q3 = st
            rs = []
            for k in range(4):
                av = ab[i, pl.ds(16 * k, 16)]
                bv = bb[i, pl.ds(H + 16 * k, 16)]
                r = jnp.maximum(av + bv, 0.0)
                rbuf[i, pl.ds(16 * k, 16)] = r
                rs.append(r)
            return (s0 + rs[0], s1 + rs[1], s2 + rs[2], s3 + rs[3],
                    q0 + rs[0] * rs[0], q1 + rs[1] * rs[1],
                    q2 + rs[2] * rs[2], q3 + rs[3] * rs[3])

        zero = jnp.zeros((16,), jnp.float32)
        cst = lax.fori_loop(0, GB, row, (zero,) * 8)
        pltpu.sync_copy(rbuf, r1_hbm.at[pl.ds(base + j * GB, GB)])
        return tuple(a + c for a, c in zip(st, cst))

    zero = jnp.zeros((16,), jnp.float32)
    st = (zero,) * 8
    fire(0, 0)
    fire(1, 1)

    def outer(j2, st):
        j = 2 * j2
        drain(0)
        st = compute(j, 0, st)
        fire(j + 2, 0)
        drain(1)
        st = compute(j + 1, 1, st)

        @pl.when(j + 3 < NCHUNK)
        def _():
            fire(j + 3, 1)

        return st

    st = lax.fori_loop(0, (NCHUNK - 1) // 2, outer, st)
    drain(0)
    st = compute(NCHUNK - 1, 0, st)

    for k in range(4):
        stat_v[0, pl.ds(16 * k, 16)] = st[k]
        stat_v[1, pl.ds(16 * k, 16)] = st[4 + k]
    pltpu.sync_copy(stat_v, st_hbm.at[wid])


def _gather_pass(T, dst, src):
    f = functools.partial(
        pl.kernel,
        out_type=[
            jax.ShapeDtypeStruct((N_EDGES, H), jnp.float32),
            jax.ShapeDtypeStruct((NW, 2, H), jnp.float32),
        ],
        mesh=_get_mesh(),
        scratch_types=[
            pltpu.VMEM((EPW,), jnp.int32),
            pltpu.VMEM((EPW,), jnp.int32),
            pltpu.VMEM((GB, 2 * H), jnp.float32),
            pltpu.VMEM((GB, 2 * H), jnp.float32),
            pltpu.VMEM((GB, 2 * H), jnp.float32),
            pltpu.VMEM((GB, 2 * H), jnp.float32),
            pltpu.VMEM((GB, H), jnp.float32),
            pltpu.VMEM((2, H), jnp.float32),
            pltpu.SemaphoreType.DMA,
            pltpu.SemaphoreType.DMA,
        ],
        compiler_params=pltpu.CompilerParams(needs_layout_passes=False),
    )(_gather_body)
    return f(T, dst, src)


def _mlp_body(wide_out, r_ref, w_ref, b_ref, o_ref, st_ref, acc):
    i = pl.program_id(0)

    @pl.when(i == 0)
    def _():
        acc[...] = jnp.zeros_like(acc)

    h = jnp.maximum(_dot(r_ref[...], w_ref[...]) + b_ref[...], 0.0)
    if wide_out:
        o_ref[:, 0:H] = h
    else:
        o_ref[...] = h
    acc[0, :] += jnp.sum(h, axis=0)
    acc[1, :] += jnp.sum(h * h, axis=0)

    @pl.when(i == pl.num_programs(0) - 1)
    def _():
        st_ref[...] = acc[...]


_MB = 8000


def _mlp_pass(r, Wt, bp, wide_out=False):
    ow = 2 * H if wide_out else H
    return pl.pallas_call(
        functools.partial(_mlp_body, wide_out),
        grid=(N_EDGES // _MB,),
        in_specs=[
            pl.BlockSpec((_MB, H), lambda i: (i, 0)),
            pl.BlockSpec((H, H), lambda i: (0, 0)),
            pl.BlockSpec((1, H), lambda i: (0, 0)),
        ],
        out_specs=[
            pl.BlockSpec((_MB, ow), lambda i: (i, 0)),
            pl.BlockSpec((2, H), lambda i: (0, 0)),
        ],
        out_shape=[
            jax.ShapeDtypeStruct((N_EDGES, ow), jnp.float32),
            jax.ShapeDtypeStruct((2, H), jnp.float32),
        ],
        scratch_shapes=[pltpu.VMEM((2, H), jnp.float32)],
    )(r, Wt, bp[None, :])


def _scatter_body(r3_hbm, dst_hbm, s3_hbm, t3_hbm, out_hbm,
                  ids0, ids1, me, gix, grow0, grow1, grow2, grow3, local, stv,
                  sem0, sem1, gsem0, gsem1, gsem2, gsem3):
    c = lax.axis_index("c")
    s = lax.axis_index("s")
    wid = s * NC + c
    lo = wid * NR
    iota = lax.iota(jnp.int32, 16)
    neg1 = jnp.full((16,), -1.0, jnp.float32)

    def initrow(i, _):
        for k in range(4):
            local[i, pl.ds(16 * k, 16)] = neg1
        return 0

    lax.fori_loop(0, NR + 1, initrow, 0)
    pltpu.sync_copy(s3_hbm, stv.at[0])
    pltpu.sync_copy(t3_hbm, stv.at[1])

    idbufs = (ids0, ids1)
    sems = (sem0, sem1)

    def fire(j, b):
        pltpu.async_copy(dst_hbm.at[pl.ds(j * DC, DC)], idbufs[b], sems[b])

    def drain(b):
        pltpu.make_async_copy(dst_hbm.at[pl.ds(0, DC)], idbufs[b], sems[b]).wait()

    def scan_chunk(j, b):
        ids = idbufs[b]
        unr = jnp.uint32(NR)

        def vreg(v, cv):
            col = ids[pl.ds(16 * v, 16)]
            u = col - lo
            m = plsc.bitcast(u, jnp.uint32) < unr
            mi = jnp.where(m, 1, 0)
            npos = cv + plsc.cumsum(mi) - 1
            eidx = j * DC + 16 * v + iota
            packed = (eidx << 9) | u
            plsc.store_scatter(me, [npos], packed, mask=m)
            return cv + plsc.all_reduce_population_count(m)

        cv = lax.fori_loop(0, DC // 16, vreg, jnp.zeros((16,), jnp.int32))
        cnt = cv[0]
        for t in range(GR // 16):
            plsc.store_scatter(me, [cnt + 16 * t + iota],
                               jnp.full((16,), -1, jnp.int32))
        return cnt

    grows = (grow0, grow1, grow2, grow3)
    gsems = (gsem0, gsem1, gsem2, gsem3)

    def fireg(g, slot):
        for t in range(GR // 16):
            mv = me[pl.ds(GR * g + 16 * t, 16)]
            ev = jnp.where(mv < 0, 0, lax.shift_right_logical(mv, 9))
            gix[slot, pl.ds(16 * t, 16)] = ev
        pltpu.async_copy(r3_hbm.at[gix.at[slot]], grows[slot], gsems[slot])

    def draing(slot):
        pltpu.make_async_copy(r3_hbm.at[gix.at[slot]], grows[slot],
                              gsems[slot]).wait()

    def update(slot, g):
        buf = grows[slot]

        def tblock(t, _):
            mv = me[pl.ds(GR * g + 16 * t, 16)]
            dv = mv & 511
            offv = jnp.where((mv >= 0) & (dv < NR), dv, NR)
            for jj in range(16):
                off_ok = offv[jj]
                row = 16 * t + jj
                for k in range(4):
                    cur = local[off_ok, pl.ds(16 * k, 16)]
                    gv = buf[row, pl.ds(16 * k, 16)]
                    local[off_ok, pl.ds(16 * k, 16)] = jnp.maximum(cur, gv)
            return 0

        lax.fori_loop(0, GR // 16, tblock, 0)

    def process(cnt):
        ng = (cnt + GR - 1) // GR
        for s4 in range(4):
            @pl.when(s4 < ng)
            def _():
                fireg(s4, s4)

        def quad(q, _):
            g = 4 * q
            for b in range(4):
                @pl.when(g + b < ng)
                def _():
                    with jax.named_scope("p5drain"):
                        draing(b)
                    with jax.named_scope("p5upd"):
                        update(b, g + b)

                    @pl.when(g + b + 4 < ng)
                    def _():
                        fireg(g + b + 4, b)

            return 0

        lax.fori_loop(0, (ng + 3) // 4, quad, 0)

    fire(0, 0)
    fire(1, 1)

    def outer(j2, _):
        j = 2 * j2
        drain(0)
        with jax.named_scope("p5scan"):
            cnt = scan_chunk(j, 0)

        @pl.when(j + 2 < NDC)
        def _():
            fire(j + 2, 0)

        process(cnt)
        drain(1)
        cnt = scan_chunk(j + 1, 1)

        @pl.when(j + 3 < NDC)
        def _():
            fire(j + 3, 1)

        process(cnt)
        return 0

    lax.fori_loop(0, NDC // 2, outer, 0)

    sv = [stv[0, pl.ds(16 * k, 16)] for k in range(4)]
    tv = [stv[1, pl.ds(16 * k, 16)] for k in range(4)]

    def finrow(i, _):
        for k in range(4):
            v = local[i, pl.ds(16 * k, 16)]
            local[i, pl.ds(16 * k, 16)] = jnp.where(
                v < 0.0, 0.0, v * sv[k] + tv[k])
        return 0

    lax.fori_loop(0, NR, finrow, 0)
    pltpu.sync_copy(local.at[pl.ds(0, NR)], out_hbm.at[pl.ds(lo, NR)])


def _scatter_pass(r3, dst, s3, t3):
    f = functools.partial(
        pl.kernel,
        out_type=jax.ShapeDtypeStruct((N_PAD, H), jnp.float32),
        mesh=_get_mesh(),
        scratch_types=[
            pltpu.VMEM((DC,), jnp.int32),
            pltpu.VMEM((DC,), jnp.int32),
            pltpu.VMEM((DC + GR,), jnp.int32),
            pltpu.VMEM((4, GR), jnp.int32),
            pltpu.VMEM((GR, 2 * H), jnp.float32),
            pltpu.VMEM((GR, 2 * H), jnp.float32),
            pltpu.VMEM((GR, 2 * H), jnp.float32),
            pltpu.VMEM((GR, 2 * H), jnp.float32),
            pltpu.VMEM((NR + 1, H), jnp.float32),
            pltpu.VMEM((2, H), jnp.float32),
            pltpu.SemaphoreType.DMA,
            pltpu.SemaphoreType.DMA,
            pltpu.SemaphoreType.DMA,
            pltpu.SemaphoreType.DMA,
            pltpu.SemaphoreType.DMA,
            pltpu.SemaphoreType.DMA,
        ],
        compiler_params=pltpu.CompilerParams(needs_layout_passes=False),
    )(_scatter_body)
    return f(r3, dst, s3, t3)


def kernel(x, pos, edge_index, W1, b1, g1, be1, W2, b2, g2, be2, W3, b3, g3, be3):
    src = edge_index[0].astype(jnp.int32)
    dst = edge_index[1].astype(jnp.int32)
    posp = jnp.pad(pos, ((0, 0), (0, 5)))
    wd = jnp.pad((W1[:, 0:3] - W1[:, 3:6]).T, ((0, 5), (0, 0)))
    wb = jnp.pad(W1[:, 3:6].T, ((0, 5), (0, 0)))
    wc = W1[:, 6:].T

    T = _tables(posp, x, wd, wb, wc, b1)
    r1, st1 = _gather_pass(T, dst, src)

    sums = jnp.sum(st1, axis=0)
    m1 = sums[0] / N_EDGES
    v1 = sums[1] / N_EDGES - m1 * m1
    a1 = g1 / jnp.sqrt(v1 + EPS)
    c1 = be1 - m1 * a1
    Wt2 = a1[:, None] * W2.T
    b2p = _dot(c1, W2.T) + b2

    r2, st2 = _mlp_pass(r1, Wt2, b2p)
    m2 = st2[0] / N_EDGES
    v2 = st2[1] / N_EDGES - m2 * m2
    a2 = g2 / jnp.sqrt(v2 + EPS)
    c2 = be2 - m2 * a2
    Wt3 = a2[:, None] * W3.T
    b3p = _dot(c2, W3.T) + b3

    r3, st3 = _mlp_pass(r2, Wt3, b3p, wide_out=True)
    m3 = st3[0] / N_EDGES
    v3 = st3[1] / N_EDGES - m3 * m3
    a3 = g3 / jnp.sqrt(v3 + EPS)
    c3 = be3 - m3 * a3

    outp = _scatter_pass(r3, dst, a3, c3)
    return outp[:N_NODES]


# speedup vs baseline: 1.7467x; 1.7467x over previous
import functools

import jax
import jax.numpy as jnp
from jax import lax
from jax.experimental import pallas as pl
from jax.experimental.pallas import tpu as pltpu
from jax.experimental.pallas import tpu_sc as plsc

N_NODES = 10000
N_EDGES = 320000
H = 64
EPS = 1e-5

NC = 2
NS = 16
NW = NC * NS
EPW = N_EDGES // NW
GB = 80
NCHUNK = EPW // GB

NR = 320
N_PAD = NW * NR
DC = 2560
NDC = N_EDGES // DC
BPW = DC // NS
GR = 64

_mesh = None


def _get_mesh():
    global _mesh
    if _mesh is None:
        _mesh = plsc.VectorSubcoreMesh(core_axis_name="c", subcore_axis_name="s")
    return _mesh


_PREC = jax.lax.Precision.HIGHEST


def _dot(a, b):
    return jnp.dot(a, b, precision=_PREC)


def _ab_body(posp_ref, x_ref, wd_ref, wb_ref, wc_ref, b1_ref, t_ref):
    p = posp_ref[...]
    t_ref[:, 0:H] = _dot(p, wd_ref[...]) + _dot(x_ref[...], wc_ref[...]) + b1_ref[...]
    t_ref[:, H:2 * H] = _dot(p, wb_ref[...])


def _tables(posp, x, wd, wb, wc, b1):
    return pl.pallas_call(
        _ab_body,
        out_shape=jax.ShapeDtypeStruct((N_NODES, 2 * H), jnp.float32),
    )(posp, x, wd, wb, wc, b1[None, :])


def _gather_body(t_hbm, dst_hbm, src_hbm, r1_hbm, st_hbm,
                 didx, sidx, ab0, ab1, bb0, bb1, rbuf, stat_v, sem0, sem1):
    c = lax.axis_index("c")
    s = lax.axis_index("s")
    wid = s * NC + c
    base = wid * EPW
    pltpu.sync_copy(dst_hbm.at[pl.ds(base, EPW)], didx)
    pltpu.sync_copy(src_hbm.at[pl.ds(base, EPW)], sidx)

    abufs = (ab0, ab1)
    bbufs = (bb0, bb1)
    sems = (sem0, sem1)

    def fire(j, b):
        pltpu.async_copy(t_hbm.at[didx.at[pl.ds(j * GB, GB)]], abufs[b], sems[b])
        pltpu.async_copy(t_hbm.at[sidx.at[pl.ds(j * GB, GB)]], bbufs[b], sems[b])

    def drain(b):
        pltpu.make_async_copy(t_hbm.at[didx.at[pl.ds(0, GB)]], abufs[b], sems[b]).wait()
        pltpu.make_async_copy(t_hbm.at[sidx.at[pl.ds(0, GB)]], bbufs[b], sems[b]).wait()

    def compute(j, b, st):
        ab = abufs[b]
        bb = bbufs[b]

        def row(i, st):
            s0, s1, s2, s3, q0, q1, q2, q3 = st
            rs = []
            for k in range(4):
                av = ab[i, pl.ds(16 * k, 16)]
                bv = bb[i, pl.ds(H + 16 * k, 16)]
                r = jnp.maximum(av + bv, 0.0)
                rbuf[i, pl.ds(16 * k, 16)] = r
                rs.append(r)
            return (s0 + rs[0], s1 + rs[1], s2 + rs[2], s3 + rs[3],
                    q0 + rs[0] * rs[0], q1 + rs[1] * rs[1],
                    q2 + rs[2] * rs[2], q3 + rs[3] * rs[3])

        zero = jnp.zeros((16,), jnp.float32)
        cst = lax.fori_loop(0, GB, row, (zero,) * 8)
        pltpu.sync_copy(rbuf, r1_hbm.at[pl.ds(base + j * GB, GB)])
        return tuple(a + c for a, c in zip(st, cst))

    zero = jnp.zeros((16,), jnp.float32)
    st = (zero,) * 8
    fire(0, 0)
    fire(1, 1)

    def outer(j2, st):
        j = 2 * j2
        drain(0)
        st = compute(j, 0, st)
        fire(j + 2, 0)
        drain(1)
        st = compute(j + 1, 1, st)

        @pl.when(j + 3 < NCHUNK)
        def _():
            fire(j + 3, 1)

        return st

    st = lax.fori_loop(0, (NCHUNK - 1) // 2, outer, st)
    drain(0)
    st = compute(NCHUNK - 1, 0, st)

    for k in range(4):
        stat_v[0, pl.ds(16 * k, 16)] = st[k]
        stat_v[1, pl.ds(16 * k, 16)] = st[4 + k]
    pltpu.sync_copy(stat_v, st_hbm.at[wid])


def _gather_pass(T, dst, src):
    f = functools.partial(
        pl.kernel,
        out_type=[
            jax.ShapeDtypeStruct((N_EDGES, H), jnp.float32),
            jax.ShapeDtypeStruct((NW, 2, H), jnp.float32),
        ],
        mesh=_get_mesh(),
        scratch_types=[
            pltpu.VMEM((EPW,), jnp.int32),
            pltpu.VMEM((EPW,), jnp.int32),
            pltpu.VMEM((GB, 2 * H), jnp.float32),
            pltpu.VMEM((GB, 2 * H), jnp.float32),
            pltpu.VMEM((GB, 2 * H), jnp.float32),
            pltpu.VMEM((GB, 2 * H), jnp.float32),
            pltpu.VMEM((GB, H), jnp.float32),
            pltpu.VMEM((2, H), jnp.float32),
            pltpu.SemaphoreType.DMA,
            pltpu.SemaphoreType.DMA,
        ],
        compiler_params=pltpu.CompilerParams(needs_layout_passes=False),
    )(_gather_body)
    return f(T, dst, src)


def _mlp_body(wide_out, r_ref, w_ref, b_ref, o_ref, st_ref, acc):
    i = pl.program_id(0)

    @pl.when(i == 0)
    def _():
        acc[...] = jnp.zeros_like(acc)

    h = jnp.maximum(_dot(r_ref[...], w_ref[...]) + b_ref[...], 0.0)
    if wide_out:
        o_ref[:, 0:H] = h
    else:
        o_ref[...] = h
    acc[0, :] += jnp.sum(h, axis=0)
    acc[1, :] += jnp.sum(h * h, axis=0)

    @pl.when(i == pl.num_programs(0) - 1)
    def _():
        st_ref[...] = acc[...]


_MB = 8000


def _mlp_pass(r, Wt, bp, wide_out=False):
    ow = 2 * H if wide_out else H
    return pl.pallas_call(
        functools.partial(_mlp_body, wide_out),
        grid=(N_EDGES // _MB,),
        in_specs=[
            pl.BlockSpec((_MB, H), lambda i: (i, 0)),
            pl.BlockSpec((H, H), lambda i: (0, 0)),
            pl.BlockSpec((1, H), lambda i: (0, 0)),
        ],
        out_specs=[
            pl.BlockSpec((_MB, ow), lambda i: (i, 0)),
            pl.BlockSpec((2, H), lambda i: (0, 0)),
        ],
        out_shape=[
            jax.ShapeDtypeStruct((N_EDGES, ow), jnp.float32),
            jax.ShapeDtypeStruct((2, H), jnp.float32),
        ],
        scratch_shapes=[pltpu.VMEM((2, H), jnp.float32)],
    )(r, Wt, bp[None, :])


def _scatter_body(r3_hbm, dst_hbm, s3_hbm, t3_hbm, out_hbm,
                  ids0, ids1, me, gix, grow0, grow1, grow2, grow3, local, stv,
                  band, sem0, sem1, gsem0, gsem1, gsem2, gsem3, bsem0, bsem1):
    c = lax.axis_index("c")
    s = lax.axis_index("s")
    wid = s * NC + c
    lo = wid * NR
    iota = lax.iota(jnp.int32, 16)
    neg1 = jnp.full((16,), -1.0, jnp.float32)

    def initrow(i, _):
        for k in range(4):
            local[i, pl.ds(16 * k, 16)] = neg1
        return 0

    lax.fori_loop(0, NR + 1, initrow, 0)
    pltpu.sync_copy(s3_hbm, stv.at[0])
    pltpu.sync_copy(t3_hbm, stv.at[1])

    idbufs = (ids0, ids1)
    sems = (sem0, sem1)
    bsems = (bsem0, bsem1)

    def fire(j, b):
        pltpu.async_copy(dst_hbm.at[pl.ds(j * DC, DC)], idbufs[b], sems[b])
        pltpu.async_copy(r3_hbm.at[pl.ds(j * DC + s * BPW, BPW)],
                         band.at[b].at[pl.ds(s * BPW, BPW)], bsems[b])

    def drain(b):
        pltpu.make_async_copy(dst_hbm.at[pl.ds(0, DC)], idbufs[b], sems[b]).wait()
        pltpu.make_async_copy(r3_hbm.at[pl.ds(0, BPW)],
                              band.at[b].at[pl.ds(0, BPW)], bsems[b]).wait()

    def scan_chunk(j, b):
        ids = idbufs[b]
        unr = jnp.uint32(NR)

        def vreg(v, cv):
            col = ids[pl.ds(16 * v, 16)]
            u = col - lo
            m = plsc.bitcast(u, jnp.uint32) < unr
            mi = jnp.where(m, 1, 0)
            npos = cv + plsc.cumsum(mi) - 1
            eidx = 16 * v + iota
            packed = (eidx << 9) | u
            plsc.store_scatter(me, [npos], packed, mask=m)
            return cv + plsc.all_reduce_population_count(m)

        cv = lax.fori_loop(0, DC // 16, vreg, jnp.zeros((16,), jnp.int32))
        cnt = cv[0]
        for t in range(GR // 16):
            plsc.store_scatter(me, [cnt + 16 * t + iota],
                               jnp.full((16,), -1, jnp.int32))
        return cnt

    grows = (grow0, grow1, grow2, grow3)
    gsems = (gsem0, gsem1, gsem2, gsem3)

    def fireg(g, slot, b):
        for t in range(GR // 16):
            mv = me[pl.ds(GR * g + 16 * t, 16)]
            ev = jnp.where(mv < 0, 0, lax.shift_right_logical(mv, 9))
            gix[slot, pl.ds(16 * t, 16)] = ev
        pltpu.async_copy(band.at[b].at[gix.at[slot]], grows[slot], gsems[slot])

    def draing(slot, b):
        pltpu.make_async_copy(band.at[b].at[gix.at[slot]], grows[slot],
                              gsems[slot]).wait()

    def update(slot, g):
        buf = grows[slot]

        def tblock(t, _):
            mv = me[pl.ds(GR * g + 16 * t, 16)]
            dv = mv & 511
            offv = jnp.where((mv >= 0) & (dv < NR), dv, NR)
            for jj in range(16):
                off_ok = offv[jj]
                row = 16 * t + jj
                for k in range(4):
                    cur = local[off_ok, pl.ds(16 * k, 16)]
                    gv = buf[row, pl.ds(16 * k, 16)]
                    local[off_ok, pl.ds(16 * k, 16)] = jnp.maximum(cur, gv)
            return 0

        lax.fori_loop(0, GR // 16, tblock, 0)

    def process(cnt, bb):
        ng = (cnt + GR - 1) // GR
        for s4 in range(4):
            @pl.when(s4 < ng)
            def _():
                fireg(s4, s4, bb)

        def quad(q, _):
            g = 4 * q
            for b in range(4):
                @pl.when(g + b < ng)
                def _():
                    draing(b, bb)
                    update(b, g + b)

                    @pl.when(g + b + 4 < ng)
                    def _():
                        fireg(g + b + 4, b, bb)

            return 0

        lax.fori_loop(0, (ng + 3) // 4, quad, 0)

    fire(0, 0)
    fire(1, 1)

    def outer(j2, _):
        j = 2 * j2
        drain(0)
        plsc.subcore_barrier()
        cnt = scan_chunk(j, 0)
        process(cnt, 0)
        plsc.subcore_barrier()

        @pl.when(j + 2 < NDC)
        def _():
            fire(j + 2, 0)

        drain(1)
        plsc.subcore_barrier()
        cnt = scan_chunk(j + 1, 1)
        process(cnt, 1)
        plsc.subcore_barrier()

        @pl.when(j + 3 < NDC)
        def _():
            fire(j + 3, 1)

        return 0

    lax.fori_loop(0, (NDC - 1) // 2, outer, 0)
    drain(0)
    plsc.subcore_barrier()
    cnt = scan_chunk(NDC - 1, 0)
    process(cnt, 0)

    sv = [stv[0, pl.ds(16 * k, 16)] for k in range(4)]
    tv = [stv[1, pl.ds(16 * k, 16)] for k in range(4)]

    def finrow(i, _):
        for k in range(4):
            v = local[i, pl.ds(16 * k, 16)]
            local[i, pl.ds(16 * k, 16)] = jnp.where(
                v < 0.0, 0.0, v * sv[k] + tv[k])
        return 0

    lax.fori_loop(0, NR, finrow, 0)
    pltpu.sync_copy(local.at[pl.ds(0, NR)], out_hbm.at[pl.ds(lo, NR)])


def _scatter_pass(r3, dst, s3, t3):
    f = functools.partial(
        pl.kernel,
        out_type=jax.ShapeDtypeStruct((N_PAD, H), jnp.float32),
        mesh=_get_mesh(),
        scratch_types=[
            pltpu.VMEM((DC,), jnp.int32),
            pltpu.VMEM((DC,), jnp.int32),
            pltpu.VMEM((DC + GR,), jnp.int32),
            pltpu.VMEM((4, GR), jnp.int32),
            pltpu.VMEM((GR, 2 * H), jnp.float32),
            pltpu.VMEM((GR, 2 * H), jnp.float32),
            pltpu.VMEM((GR, 2 * H), jnp.float32),
            pltpu.VMEM((GR, 2 * H), jnp.float32),
            pltpu.VMEM((NR + 1, H), jnp.float32),
            pltpu.VMEM((2, H), jnp.float32),
            pltpu.VMEM_SHARED((2, DC, 2 * H), jnp.float32),
            pltpu.SemaphoreType.DMA,
            pltpu.SemaphoreType.DMA,
            pltpu.SemaphoreType.DMA,
            pltpu.SemaphoreType.DMA,
            pltpu.SemaphoreType.DMA,
            pltpu.SemaphoreType.DMA,
            pltpu.SemaphoreType.DMA,
            pltpu.SemaphoreType.DMA,
        ],
        compiler_params=pltpu.CompilerParams(needs_layout_passes=False),
    )(_scatter_body)
    return f(r3, dst, s3, t3)


def kernel(x, pos, edge_index, W1, b1, g1, be1, W2, b2, g2, be2, W3, b3, g3, be3):
    src = edge_index[0].astype(jnp.int32)
    dst = edge_index[1].astype(jnp.int32)
    posp = jnp.pad(pos, ((0, 0), (0, 5)))
    wd = jnp.pad((W1[:, 0:3] - W1[:, 3:6]).T, ((0, 5), (0, 0)))
    wb = jnp.pad(W1[:, 3:6].T, ((0, 5), (0, 0)))
    wc = W1[:, 6:].T

    T = _tables(posp, x, wd, wb, wc, b1)
    r1, st1 = _gather_pass(T, dst, src)

    sums = jnp.sum(st1, axis=0)
    m1 = sums[0] / N_EDGES
    v1 = sums[1] / N_EDGES - m1 * m1
    a1 = g1 / jnp.sqrt(v1 + EPS)
    c1 = be1 - m1 * a1
    Wt2 = a1[:, None] * W2.T
    b2p = _dot(c1, W2.T) + b2

    r2, st2 = _mlp_pass(r1, Wt2, b2p)
    m2 = st2[0] / N_EDGES
    v2 = st2[1] / N_EDGES - m2 * m2
    a2 = g2 / jnp.sqrt(v2 + EPS)
    c2 = be2 - m2 * a2
    Wt3 = a2[:, None] * W3.T
    b3p = _dot(c2, W3.T) + b3

    r3, st3 = _mlp_pass(r2, Wt3, b3p, wide_out=True)
    m3 = st3[0] / N_EDGES
    v3 = st3[1] / N_EDGES - m3 * m3
    a3 = g3 / jnp.sqrt(v3 + EPS)
    c3 = be3 - m3 * a3

    outp = _scatter_pass(r3, dst, a3, c3)
    return outp[:N_NODES]
